# baseline (speedup 1.0000x reference)
import functools

import jax
import jax.numpy as jnp
from jax import lax
from jax.experimental import pallas as pl
from jax.experimental.pallas import tpu as pltpu
from jax.experimental.pallas import tpu_sc as plsc

_INCREASE_DELTA = 0.05
_ABS_THRESHOLD = 0.8

_B = 16
_S = 4096
_L = 16
_NCHUNK = _S // _L
_NQ = 4
_QCHUNK = _NCHUNK // _NQ
_QLEN = _S // _NQ
_PAD = _L
_NEG = -3e38
_POS = 3e38


def _seg_body(ent_hbm, seg_hbm, pem_hbm, fb_hbm, row_v, seg_v, pem_v, fb_v):
    wid = lax.axis_index("s")

    rb = wid * _S
    row_v[pl.ds(0, _L)] = jnp.full((_L,), _NEG, jnp.float32)
    pltpu.sync_copy(ent_hbm.at[pl.ds(rb, _S)], row_v.at[pl.ds(_PAD, _S)])
    row_v[pl.ds(_PAD + _S, _L)] = jnp.full((_L,), _POS, jnp.float32)

    lane = lax.iota(jnp.int32, _L)
    zeros = jnp.zeros((_L,), jnp.int32)

    def chunk(i, carry_sum, carry_max):
        base = _PAD + i * _L
        prev = row_v[pl.ds(base - 1, _L)]
        e = row_v[pl.ds(base, _L)]
        nxt = row_v[pl.ds(base + 1, _L)]
        inc = (e > prev + _INCREASE_DELTA) | (e > _ABS_THRESHOLD)
        pem = (nxt > e + _INCREASE_DELTA) | (nxt > _ABS_THRESHOLD)
        inc_i = inc.astype(jnp.int32)
        off = i * _L
        seg_v[pl.ds(off, _L)] = plsc.cumsum(inc_i) + carry_sum
        pem_v[pl.ds(off, _L)] = pem.astype(jnp.int32)
        fp = jnp.where(inc, off + lane, 0)
        fb_v[pl.ds(off, _L)] = jnp.maximum(plsc.cummax(fp), carry_max)
        cnt = plsc.all_reduce_population_count(inc)
        ffs = plsc.all_reduce_ffs(lax.rev(inc_i, (0,)) != 0)
        last_pos = (off + 15) - ffs
        new_max = jnp.where(cnt > 0, last_pos, carry_max)
        return carry_sum + cnt, new_max

    def step(k, carry):
        new = []
        for q in range(_NQ):
            cs, cm = carry[2 * q], carry[2 * q + 1]
            cs, cm = chunk(q * _QCHUNK + k, cs, cm)
            new += [cs, cm]
        return tuple(new)

    carries = lax.fori_loop(0, _QCHUNK, step, (zeros, zeros) * _NQ)

    sum_off = zeros - 1
    max_off = zeros
    offs = []
    for q in range(_NQ):
        offs.append((sum_off, max_off))
        sum_off = sum_off + carries[2 * q]
        max_off = jnp.maximum(max_off, carries[2 * q + 1])

    def fix(k, c):
        for q in range(1, _NQ):
            off = (q * _QCHUNK + k) * _L
            s_off, m_off = offs[q]
            seg_v[pl.ds(off, _L)] = seg_v[pl.ds(off, _L)] + s_off
            fb_v[pl.ds(off, _L)] = jnp.maximum(fb_v[pl.ds(off, _L)], m_off)
        off0 = k * _L
        seg_v[pl.ds(off0, _L)] = seg_v[pl.ds(off0, _L)] + offs[0][0]
        return c

    lax.fori_loop(0, _QCHUNK, fix, 0)

    pltpu.sync_copy(seg_v, seg_hbm.at[pl.ds(rb, _S)])
    pltpu.sync_copy(pem_v, pem_hbm.at[pl.ds(rb, _S)])
    pltpu.sync_copy(fb_v, fb_hbm.at[pl.ds(rb, _S)])


@jax.jit
def _segmenter(entropy_bits):
    mesh = plsc.VectorSubcoreMesh(
        core_axis_name="c", subcore_axis_name="s", num_cores=1, num_subcores=16
    )
    out = jax.ShapeDtypeStruct((_B * _S,), jnp.int32)
    run = functools.partial(
        pl.kernel,
        out_type=(out, out, out),
        mesh=mesh,
        compiler_params=pltpu.CompilerParams(
            needs_layout_passes=False, skip_device_barrier=True
        ),
        scratch_types=[
            pltpu.VMEM((_PAD + _S + _L,), jnp.float32),
            pltpu.VMEM((_S,), jnp.int32),
            pltpu.VMEM((_S,), jnp.int32),
            pltpu.VMEM((_S,), jnp.int32),
        ],
    )(_seg_body)
    seg, pem, fb = run(entropy_bits.reshape(_B * _S))
    return (
        seg.reshape(_B, _S),
        pem.reshape(_B, _S) != 0,
        fb.reshape(_B, _S),
    )


def kernel(entropy_bits):
    return _segmenter(entropy_bits)


# speedup vs baseline: 1.0664x; 1.0664x over previous
import functools

import jax
import jax.numpy as jnp
from jax import lax
from jax.experimental import pallas as pl
from jax.experimental.pallas import tpu as pltpu
from jax.experimental.pallas import tpu_sc as plsc

_INCREASE_DELTA = 0.05
_ABS_THRESHOLD = 0.8

_B = 16
_S = 4096
_L = 16
_NCHUNK = _S // _L
_PAD = _L
_NEG = -3e38
_POS = 3e38


def _seg_body(ent_hbm, seg_hbm, fb_hbm, row_v, seg_v, fb_v):
    wid = lax.axis_index("s")

    rb = wid * _S
    row_v[pl.ds(0, _L)] = jnp.full((_L,), _NEG, jnp.float32)
    pltpu.sync_copy(ent_hbm.at[pl.ds(rb, _S)], row_v.at[pl.ds(_PAD, _S)])

    lane = lax.iota(jnp.int32, _L)

    def chunk(i, carry):
        carry_sum, carry_max = carry
        base = _PAD + i * _L
        prev = row_v[pl.ds(base - 1, _L)]
        e = row_v[pl.ds(base, _L)]
        inc = (e > prev + _INCREASE_DELTA) | (e > _ABS_THRESHOLD)
        inc_i = inc.astype(jnp.int32)
        off = i * _L
        seg_v[pl.ds(off, _L)] = plsc.cumsum(inc_i) + (carry_sum - 1)
        fp = jnp.where(inc, off + lane, 0)
        fb_v[pl.ds(off, _L)] = jnp.maximum(plsc.cummax(fp), carry_max)
        cnt = plsc.all_reduce_population_count(inc)
        ffs = plsc.all_reduce_ffs(lax.rev(inc_i, (0,)) != 0)
        new_max = jnp.where(cnt > 0, (off + 15) - ffs, carry_max)
        return carry_sum + cnt, new_max

    lax.fori_loop(
        0, _NCHUNK, chunk,
        (jnp.zeros((_L,), jnp.int32), jnp.zeros((_L,), jnp.int32)),
    )

    pltpu.sync_copy(seg_v, seg_hbm.at[pl.ds(rb, _S)])
    pltpu.sync_copy(fb_v, fb_hbm.at[pl.ds(rb, _S)])


def _pem_body(ent_ref, pem_ref):
    e = ent_ref[...]
    nxt = jnp.concatenate(
        [e[:, 1:], jnp.full((_B, 1), _POS, jnp.float32)], axis=1
    )
    pem_ref[...] = (nxt > e + _INCREASE_DELTA) | (nxt > _ABS_THRESHOLD)


@jax.jit
def _segmenter(entropy_bits):
    mesh = plsc.VectorSubcoreMesh(
        core_axis_name="c", subcore_axis_name="s", num_cores=1, num_subcores=16
    )
    out = jax.ShapeDtypeStruct((_B * _S,), jnp.int32)
    run = functools.partial(
        pl.kernel,
        out_type=(out, out),
        mesh=mesh,
        compiler_params=pltpu.CompilerParams(
            needs_layout_passes=False, skip_device_barrier=True
        ),
        scratch_types=[
            pltpu.VMEM((_PAD + _S,), jnp.float32),
            pltpu.VMEM((_S,), jnp.int32),
            pltpu.VMEM((_S,), jnp.int32),
        ],
    )(_seg_body)
    seg, fb = run(entropy_bits.reshape(_B * _S))
    pem = pl.pallas_call(
        _pem_body,
        out_shape=jax.ShapeDtypeStruct((_B, _S), jnp.bool_),
    )(entropy_bits)
    return seg.reshape(_B, _S), pem, fb.reshape(_B, _S)


def kernel(entropy_bits):
    return _segmenter(entropy_bits)


# speedup vs baseline: 1.1048x; 1.0360x over previous
import functools

import jax
import jax.numpy as jnp
from jax import lax
from jax.experimental import pallas as pl
from jax.experimental.pallas import tpu as pltpu
from jax.experimental.pallas import tpu_sc as plsc

_INCREASE_DELTA = 0.05
_ABS_THRESHOLD = 0.8

_B = 16
_S = 4096
_L = 16
_NCHUNK = _S // _L
_PAD = _L
_NEG = -3e38
_POS = 3e38


def _seg_body(ent_hbm, res_hbm, row_v, res_v):
    wid = lax.axis_index("s")

    rb = wid * _S
    row_v[pl.ds(0, _L)] = jnp.full((_L,), _NEG, jnp.float32)
    pltpu.sync_copy(ent_hbm.at[pl.ds(rb, _S)], row_v.at[pl.ds(_PAD, _S)])

    lane = lax.iota(jnp.int32, _L)

    def chunk(i, carry):
        carry_sum, carry_max = carry
        base = _PAD + i * _L
        prev = row_v[pl.ds(base - 1, _L)]
        e = row_v[pl.ds(base, _L)]
        inc = (e > prev + _INCREASE_DELTA) | (e > _ABS_THRESHOLD)
        inc_i = inc.astype(jnp.int32)
        off = i * _L
        res_v[pl.ds(off, _L)] = plsc.cumsum(inc_i) + carry_sum
        fp = jnp.where(inc, off + lane, 0)
        res_v[pl.ds(_S + off, _L)] = jnp.maximum(plsc.cummax(fp), carry_max)
        cnt = plsc.all_reduce_population_count(inc)
        ffs = plsc.all_reduce_ffs(lax.rev(inc_i, (0,)) != 0)
        new_max = jnp.where(cnt > 0, (off + 15) - ffs, carry_max)
        return carry_sum + cnt, new_max

    lax.fori_loop(
        0, _NCHUNK, chunk,
        (jnp.full((_L,), -1, jnp.int32), jnp.zeros((_L,), jnp.int32)),
    )

    pltpu.sync_copy(res_v, res_hbm.at[pl.ds(wid * (2 * _S), 2 * _S)])


def _stage_in_body(ent_ref, pem_ref, flat_ref):
    e = ent_ref[...]
    nxt = jnp.concatenate(
        [e[:, 1:], jnp.full((_B, 1), _POS, jnp.float32)], axis=1
    )
    pem_ref[...] = (nxt > e + _INCREASE_DELTA) | (nxt > _ABS_THRESHOLD)
    flat_ref[...] = e.reshape(_B * _S)


def _stage_out_body(res_ref, seg_ref, fb_ref):
    x = res_ref[...].reshape(_B, 2, _S)
    seg_ref[...] = x[:, 0, :]
    fb_ref[...] = x[:, 1, :]


@jax.jit
def _segmenter(entropy_bits):
    pem, ent_flat = pl.pallas_call(
        _stage_in_body,
        out_shape=(
            jax.ShapeDtypeStruct((_B, _S), jnp.bool_),
            jax.ShapeDtypeStruct((_B * _S,), jnp.float32),
        ),
    )(entropy_bits)

    mesh = plsc.VectorSubcoreMesh(
        core_axis_name="c", subcore_axis_name="s", num_cores=1, num_subcores=16
    )
    run = functools.partial(
        pl.kernel,
        out_type=jax.ShapeDtypeStruct((_B * 2 * _S,), jnp.int32),
        mesh=mesh,
        compiler_params=pltpu.CompilerParams(
            needs_layout_passes=False, skip_device_barrier=True
        ),
        scratch_types=[
            pltpu.VMEM((_PAD + _S,), jnp.float32),
            pltpu.VMEM((2 * _S,), jnp.int32),
        ],
    )(_seg_body)
    res = run(ent_flat)

    seg, fb = pl.pallas_call(
        _stage_out_body,
        out_shape=(
            jax.ShapeDtypeStruct((_B, _S), jnp.int32),
            jax.ShapeDtypeStruct((_B, _S), jnp.int32),
        ),
    )(res)
    return seg, pem, fb


def kernel(entropy_bits):
    return _segmenter(entropy_bits)


# speedup vs baseline: 1.1114x; 1.0059x over previous
import functools

import jax
import jax.numpy as jnp
from jax import lax
from jax.experimental import pallas as pl
from jax.experimental.pallas import tpu as pltpu
from jax.experimental.pallas import tpu_sc as plsc

_INCREASE_DELTA = 0.05
_ABS_THRESHOLD = 0.8

_B = 16
_S = 4096
_L = 16
_NCHUNK = _S // _L
_PAD = _L
_NEG = -3e38
_POS = 3e38


def _seg_body(ent_hbm, res_hbm, row_v, res_v):
    wid = lax.axis_index("s")

    rb = wid * _S
    row_v[pl.ds(0, _L)] = jnp.full((_L,), _NEG, jnp.float32)
    pltpu.sync_copy(ent_hbm.at[pl.ds(rb, _S)], row_v.at[pl.ds(_PAD, _S)])

    lane = lax.iota(jnp.int32, _L)

    def chunk(i, carry):
        carry_sum, carry_max = carry
        base = _PAD + i * _L
        prev = row_v[pl.ds(base - 1, _L)]
        e = row_v[pl.ds(base, _L)]
        inc = (e > prev + _INCREASE_DELTA) | (e > _ABS_THRESHOLD)
        inc_i = inc.astype(jnp.int32)
        off = i * _L
        res_v[pl.ds(off, _L)] = plsc.cumsum(inc_i) + carry_sum
        fp = jnp.where(inc, off + lane, 0)
        res_v[pl.ds(_S + off, _L)] = jnp.maximum(plsc.cummax(fp), carry_max)
        cnt = plsc.all_reduce_population_count(inc)
        ffs = plsc.all_reduce_ffs(lax.rev(inc_i, (0,)) != 0)
        new_max = jnp.where(cnt > 0, (off + 15) - ffs, carry_max)
        return carry_sum + cnt, new_max

    lax.fori_loop(
        0, _NCHUNK, chunk,
        (jnp.full((_L,), -1, jnp.int32), jnp.zeros((_L,), jnp.int32)),
    )

    pltpu.sync_copy(res_v, res_hbm.at[pl.ds(wid * (2 * _S), 2 * _S)])


def _stage_in_body(ent_ref, pem_ref, flat_ref):
    e = ent_ref[...]
    nxt = jnp.concatenate(
        [e[:, 1:], jnp.full((_B, 1), _POS, jnp.float32)], axis=1
    )
    pem_ref[...] = (nxt > e + _INCREASE_DELTA) | (nxt > _ABS_THRESHOLD)
    flat_ref[...] = e.reshape(_B * _S)


def _stage_out_body(res_ref, seg_ref, fb_ref):
    x = res_ref[...].reshape(8, 2, _S)
    seg_ref[...] = x[:, 0, :]
    fb_ref[...] = x[:, 1, :]


@jax.jit
def _segmenter(entropy_bits):
    pem, ent_flat = pl.pallas_call(
        _stage_in_body,
        out_shape=(
            jax.ShapeDtypeStruct((_B, _S), jnp.bool_),
            jax.ShapeDtypeStruct((_B * _S,), jnp.float32),
        ),
    )(entropy_bits)

    mesh = plsc.VectorSubcoreMesh(
        core_axis_name="c", subcore_axis_name="s", num_cores=1, num_subcores=16
    )
    run = functools.partial(
        pl.kernel,
        out_type=jax.ShapeDtypeStruct((_B * 2 * _S,), jnp.int32),
        mesh=mesh,
        compiler_params=pltpu.CompilerParams(
            needs_layout_passes=False, skip_device_barrier=True
        ),
        scratch_types=[
            pltpu.VMEM((_PAD + _S,), jnp.float32),
            pltpu.VMEM((2 * _S,), jnp.int32),
        ],
    )(_seg_body)
    res = run(ent_flat)

    seg, fb = pl.pallas_call(
        _stage_out_body,
        grid=(2,),
        in_specs=[pl.BlockSpec((8 * 2 * _S,), lambda w: (w,))],
        out_specs=(
            pl.BlockSpec((8, _S), lambda w: (w, 0)),
            pl.BlockSpec((8, _S), lambda w: (w, 0)),
        ),
        out_shape=(
            jax.ShapeDtypeStruct((_B, _S), jnp.int32),
            jax.ShapeDtypeStruct((_B, _S), jnp.int32),
        ),
    )(res)
    return seg, pem, fb


def kernel(entropy_bits):
    return _segmenter(entropy_bits)
